# Initial kernel scaffold; baseline (speedup 1.0000x reference)
import jax
import jax.numpy as jnp
from jax.experimental import pallas as pl


def kernel(inputs, embedding):
    raise NotImplementedError("write your pallas kernel here")


# speedup vs baseline: 1.8521x; 1.8521x over previous
import functools

import jax
import jax.numpy as jnp
from jax.experimental import pallas as pl
from jax.experimental.pallas import tpu as pltpu
from jax.experimental.pallas import tpu_sc as plsc

COMMITMENT_COST = 0.25

N = 8192
C = 8192
D = 256

TN = 512
TC = 2048


def _vq_body(x_ref, emb_ref, x2_ref, e2_ref, dist_ref, idx_ref, loss_ref,
             rmax_ref, rarg_ref, acc_ref):
    i = pl.program_id(0)
    j = pl.program_id(1)
    ni = pl.num_programs(0)
    nj = pl.num_programs(1)

    x = x_ref[...]
    emb = emb_ref[pl.ds(j * TC, TC), :]

    xy = jax.lax.dot_general(
        x, emb, (((1,), (1,)), ((), ())),
        preferred_element_type=jnp.float32)
    x2 = x2_ref[...]
    e2 = e2_ref[0]
    d2 = jnp.maximum(x2 + e2 - 2.0 * xy, 0.0)
    vals = -jnp.sqrt(d2)
    dist_ref[...] = vals

    bmax = jnp.max(vals, axis=1, keepdims=True)
    col = jax.lax.broadcasted_iota(jnp.int32, (TN, TC), 1) + j * TC
    barg = jnp.min(jnp.where(vals == bmax, col, jnp.int32(C)),
                   axis=1, keepdims=True)

    @pl.when(j == 0)
    def _():
        rmax_ref[...] = bmax
        rarg_ref[...] = barg

    @pl.when(j > 0)
    def _():
        upd = bmax > rmax_ref[...]
        rmax_ref[...] = jnp.where(upd, bmax, rmax_ref[...])
        rarg_ref[...] = jnp.where(upd, barg, rarg_ref[...])

    @pl.when(jnp.logical_and(i == 0, j == 0))
    def _():
        acc_ref[0, 0] = 0.0

    @pl.when(j == nj - 1)
    def _():
        idx_ref[...] = rarg_ref[...]
        m = rmax_ref[...]
        acc_ref[0, 0] += jnp.sum(m * m)

    @pl.when(jnp.logical_and(i == ni - 1, j == nj - 1))
    def _():
        loss_ref[0, 0] = acc_ref[0, 0] * (COMMITMENT_COST / (N * D))


def _vq_distances(xt, emb, x2, e2):
    grid = (N // TN, C // TC)
    return pl.pallas_call(
        _vq_body,
        grid=grid,
        in_specs=[
            pl.BlockSpec((TN, D), lambda i, j: (i, 0)),
            pl.BlockSpec((C, D), lambda i, j: (0, 0)),
            pl.BlockSpec((TN, 1), lambda i, j: (i, 0)),
            pl.BlockSpec((1, 1, TC), lambda i, j: (0, 0, j)),
        ],
        out_specs=[
            pl.BlockSpec((TN, TC), lambda i, j: (i, j)),
            pl.BlockSpec((TN, 1), lambda i, j: (i, 0)),
            pl.BlockSpec(memory_space=pltpu.SMEM),
        ],
        out_shape=[
            jax.ShapeDtypeStruct((N, C), jnp.float32),
            jax.ShapeDtypeStruct((N, 1), jnp.int32),
            jax.ShapeDtypeStruct((1, 1), jnp.float32),
        ],
        scratch_shapes=[
            pltpu.VMEM((TN, 1), jnp.float32),
            pltpu.VMEM((TN, 1), jnp.int32),
            pltpu.SMEM((1, 1), jnp.float32),
        ],
    )(xt, emb, x2, e2)


def _sc_gather(emb, idx):
    info = plsc.get_sparse_core_info()
    nworkers = info.num_cores * info.num_subcores
    bpw = N // nworkers
    mesh = plsc.VectorSubcoreMesh(core_axis_name="c", subcore_axis_name="s")

    @functools.partial(
        pl.kernel, mesh=mesh,
        out_type=jax.ShapeDtypeStruct((N, D), jnp.float32),
        scratch_types=[
            pltpu.VMEM((bpw,), jnp.int32),
            pltpu.VMEM((bpw, D), jnp.float32),
            pltpu.SemaphoreType.DMA,
        ],
    )
    def k(emb_hbm, idx_hbm, out_hbm, idx_v, rows_v, sem):
        wid = jax.lax.axis_index("s") * info.num_cores + jax.lax.axis_index("c")
        base = wid * bpw
        pltpu.sync_copy(idx_hbm.at[pl.ds(base, bpw)], idx_v)
        pltpu.async_copy(emb_hbm.at[idx_v], rows_v, sem).wait()
        pltpu.sync_copy(rows_v, out_hbm.at[pl.ds(base, bpw)])

    return k(emb, idx)


def kernel(inputs, embedding):
    x = jnp.transpose(inputs, (0, 2, 1))
    xt = x[0]
    emb = embedding[0]
    x2 = jnp.sum(x * x, axis=-1, keepdims=True)[0]
    e2 = jnp.sum(embedding * embedding, axis=-1)[:, None, :]

    dist, idxc, lossc = _vq_distances(xt, emb, x2, e2)
    idx_flat = idxc.reshape(N)
    quant = _sc_gather(emb, idx_flat)

    out = jnp.transpose(quant)[None]
    encoding_indices = idx_flat[None]
    loss = lossc.reshape(())
    distances = dist[None]
    return (out, encoding_indices, loss, distances)


# speedup vs baseline: 1.8645x; 1.0066x over previous
import functools

import jax
import jax.numpy as jnp
from jax.experimental import pallas as pl
from jax.experimental.pallas import tpu as pltpu
from jax.experimental.pallas import tpu_sc as plsc

COMMITMENT_COST = 0.25

N = 8192
C = 8192
D = 256

TN = 512
TC = 2048


def _vq_body(x_ref, emb_ref, x2_ref, e2_ref, dist_ref, idx_ref, loss_ref,
             rmax_ref, rarg_ref, acc_ref):
    i = pl.program_id(0)
    j = pl.program_id(1)
    ni = pl.num_programs(0)
    nj = pl.num_programs(1)

    x = x_ref[...]
    emb = emb_ref[pl.ds(j * TC, TC), :]

    xy = jax.lax.dot_general(
        x, emb, (((1,), (1,)), ((), ())),
        preferred_element_type=jnp.float32)
    x2 = x2_ref[...]
    e2 = e2_ref[0]
    d2 = jnp.maximum(x2 + e2 - 2.0 * xy, 0.0)
    vals = -jnp.sqrt(d2)
    dist_ref[...] = vals

    bmax = jnp.max(vals, axis=1, keepdims=True)
    col = jax.lax.broadcasted_iota(jnp.int32, (TN, TC), 1) + j * TC
    barg = jnp.min(jnp.where(vals == bmax, col, jnp.int32(C)),
                   axis=1, keepdims=True)

    @pl.when(j == 0)
    def _():
        rmax_ref[...] = bmax
        rarg_ref[...] = barg

    @pl.when(j > 0)
    def _():
        upd = bmax > rmax_ref[...]
        rmax_ref[...] = jnp.where(upd, bmax, rmax_ref[...])
        rarg_ref[...] = jnp.where(upd, barg, rarg_ref[...])

    @pl.when(jnp.logical_and(i == 0, j == 0))
    def _():
        acc_ref[0, 0] = 0.0

    @pl.when(j == nj - 1)
    def _():
        idx_ref[...] = rarg_ref[...].reshape(TN)
        m = rmax_ref[...]
        acc_ref[0, 0] += jnp.sum(m * m)

    @pl.when(jnp.logical_and(i == ni - 1, j == nj - 1))
    def _():
        loss_ref[0, 0] = acc_ref[0, 0] * (COMMITMENT_COST / (N * D))


def _vq_distances(xt, emb, x2, e2):
    grid = (N // TN, C // TC)
    return pl.pallas_call(
        _vq_body,
        grid=grid,
        in_specs=[
            pl.BlockSpec((TN, D), lambda i, j: (i, 0)),
            pl.BlockSpec((C, D), lambda i, j: (0, 0)),
            pl.BlockSpec((TN, 1), lambda i, j: (i, 0)),
            pl.BlockSpec((1, 1, TC), lambda i, j: (0, 0, j)),
        ],
        out_specs=[
            pl.BlockSpec((TN, TC), lambda i, j: (i, j)),
            pl.BlockSpec((TN,), lambda i, j: (i,)),
            pl.BlockSpec(memory_space=pltpu.SMEM),
        ],
        out_shape=[
            jax.ShapeDtypeStruct((N, C), jnp.float32),
            jax.ShapeDtypeStruct((N,), jnp.int32),
            jax.ShapeDtypeStruct((1, 1), jnp.float32),
        ],
        scratch_shapes=[
            pltpu.VMEM((TN, 1), jnp.float32),
            pltpu.VMEM((TN, 1), jnp.int32),
            pltpu.SMEM((1, 1), jnp.float32),
        ],
    )(xt, emb, x2, e2)


def _sc_gather(emb, idx):
    info = plsc.get_sparse_core_info()
    nworkers = info.num_cores * info.num_subcores
    bpw = N // nworkers
    mesh = plsc.VectorSubcoreMesh(core_axis_name="c", subcore_axis_name="s")

    @functools.partial(
        pl.kernel, mesh=mesh,
        out_type=jax.ShapeDtypeStruct((N, D), jnp.float32),
        scratch_types=[
            pltpu.VMEM((bpw,), jnp.int32),
            pltpu.VMEM((bpw, D), jnp.float32),
            pltpu.SemaphoreType.DMA,
        ],
    )
    def k(emb_hbm, idx_hbm, out_hbm, idx_v, rows_v, sem):
        wid = jax.lax.axis_index("s") * info.num_cores + jax.lax.axis_index("c")
        base = wid * bpw
        pltpu.sync_copy(idx_hbm.at[pl.ds(base, bpw)], idx_v)
        pltpu.async_copy(emb_hbm.at[idx_v], rows_v, sem).wait()
        pltpu.sync_copy(rows_v, out_hbm.at[pl.ds(base, bpw)])

    return k(emb, idx)


def kernel(inputs, embedding):
    x = jnp.transpose(inputs, (0, 2, 1))
    xt = x[0]
    emb = embedding[0]
    x2 = jnp.sum(x * x, axis=-1, keepdims=True)[0]
    e2 = jnp.sum(embedding * embedding, axis=-1)[:, None, :]

    dist, idx_flat, lossc = _vq_distances(xt, emb, x2, e2)
    quant = _sc_gather(emb, idx_flat)

    out = jnp.transpose(quant)[None]
    encoding_indices = idx_flat[None]
    loss = lossc.reshape(())
    distances = dist[None]
    return (out, encoding_indices, loss, distances)


# speedup vs baseline: 2.0404x; 1.0944x over previous
import functools

import jax
import jax.numpy as jnp
from jax.experimental import pallas as pl
from jax.experimental.pallas import tpu as pltpu
from jax.experimental.pallas import tpu_sc as plsc

COMMITMENT_COST = 0.25

N = 8192
C = 8192
D = 256

TN = 512
NI = N // TN


def _vq_body(x_ref, emb_ref, x2_ref, e2_ref, dist_ref, idx_ref, lossp_ref):
    x = x_ref[...]
    emb = emb_ref[...]

    xy = jax.lax.dot_general(
        x, emb, (((1,), (1,)), ((), ())),
        preferred_element_type=jnp.float32)
    x2 = x2_ref[...]
    e2 = e2_ref[0]
    d2 = jnp.maximum(x2 + e2 - 2.0 * xy, 0.0)
    vals = -jnp.sqrt(d2)
    dist_ref[...] = vals

    bmax = jnp.max(vals, axis=1, keepdims=True)
    col = jax.lax.broadcasted_iota(jnp.int32, (TN, C), 1)
    barg = jnp.min(jnp.where(vals == bmax, col, jnp.int32(C)),
                   axis=1, keepdims=True)
    idx_ref[...] = barg.reshape(TN)
    lossp_ref[0, 0, 0] = jnp.sum(bmax * bmax)


def _vq_distances(xt, emb, x2, e2):
    return pl.pallas_call(
        _vq_body,
        grid=(NI,),
        in_specs=[
            pl.BlockSpec((TN, D), lambda i: (i, 0)),
            pl.BlockSpec((C, D), lambda i: (0, 0)),
            pl.BlockSpec((TN, 1), lambda i: (i, 0)),
            pl.BlockSpec((1, 1, C), lambda i: (0, 0, 0)),
        ],
        out_specs=[
            pl.BlockSpec((TN, C), lambda i: (i, 0)),
            pl.BlockSpec((TN,), lambda i: (i,)),
            pl.BlockSpec((1, 1, 1), lambda i: (i, 0, 0), memory_space=pltpu.SMEM),
        ],
        out_shape=[
            jax.ShapeDtypeStruct((N, C), jnp.float32),
            jax.ShapeDtypeStruct((N,), jnp.int32),
            jax.ShapeDtypeStruct((NI, 1, 1), jnp.float32),
        ],
    )(xt, emb, x2, e2)


def _sc_gather(emb, idx):
    info = plsc.get_sparse_core_info()
    nworkers = info.num_cores * info.num_subcores
    bpw = N // nworkers
    mesh = plsc.VectorSubcoreMesh(core_axis_name="c", subcore_axis_name="s")

    @functools.partial(
        pl.kernel, mesh=mesh,
        out_type=jax.ShapeDtypeStruct((N, D), jnp.float32),
        scratch_types=[
            pltpu.VMEM((bpw,), jnp.int32),
            pltpu.VMEM((bpw, D), jnp.float32),
            pltpu.SemaphoreType.DMA,
        ],
    )
    def k(emb_hbm, idx_hbm, out_hbm, idx_v, rows_v, sem):
        wid = jax.lax.axis_index("s") * info.num_cores + jax.lax.axis_index("c")
        base = wid * bpw
        pltpu.sync_copy(idx_hbm.at[pl.ds(base, bpw)], idx_v)
        pltpu.async_copy(emb_hbm.at[idx_v], rows_v, sem).wait()
        pltpu.sync_copy(rows_v, out_hbm.at[pl.ds(base, bpw)])

    return k(emb, idx)


def kernel(inputs, embedding):
    x = jnp.transpose(inputs, (0, 2, 1))
    xt = x[0]
    emb = embedding[0]
    x2 = jnp.sum(x * x, axis=-1, keepdims=True)[0]
    e2 = jnp.sum(embedding * embedding, axis=-1)[:, None, :]

    dist, idx_flat, lossp = _vq_distances(xt, emb, x2, e2)
    quant = _sc_gather(emb, idx_flat)

    out = jnp.transpose(quant)[None]
    encoding_indices = idx_flat[None]
    loss = (jnp.sum(lossp) * (COMMITMENT_COST / (N * D))).reshape(())
    distances = dist[None]
    return (out, encoding_indices, loss, distances)


# speedup vs baseline: 2.0432x; 1.0014x over previous
import functools

import jax
import jax.numpy as jnp
from jax.experimental import pallas as pl
from jax.experimental.pallas import tpu as pltpu
from jax.experimental.pallas import tpu_sc as plsc

COMMITMENT_COST = 0.25

N = 8192
C = 8192
D = 256

TN = 512
NI = N // TN


def _vq_body(x_ref, emb_ref, x2_ref, e2_ref, dist_ref, idx_ref, lossp_ref):
    x = x_ref[...]
    emb = emb_ref[...]

    xy = jax.lax.dot_general(
        x, emb, (((1,), (1,)), ((), ())),
        preferred_element_type=jnp.float32)
    x2 = x2_ref[...]
    e2 = e2_ref[0]
    d2 = jnp.maximum(x2 + e2 - 2.0 * xy, 0.0)
    vals = -jnp.sqrt(d2)
    dist_ref[...] = vals

    bmax = jnp.max(vals, axis=1, keepdims=True)
    col = jax.lax.broadcasted_iota(jnp.int32, (TN, C), 1)
    barg = jnp.min(jnp.where(vals == bmax, col, jnp.int32(C)),
                   axis=1, keepdims=True)
    idx_ref[...] = barg.reshape(TN)
    lossp_ref[0, 0, 0] = jnp.sum(bmax * bmax)


def _vq_distances(xt, emb, x2, e2):
    return pl.pallas_call(
        _vq_body,
        grid=(NI,),
        compiler_params=pltpu.CompilerParams(
            dimension_semantics=("parallel",)),
        in_specs=[
            pl.BlockSpec((TN, D), lambda i: (i, 0)),
            pl.BlockSpec((C, D), lambda i: (0, 0)),
            pl.BlockSpec((TN, 1), lambda i: (i, 0)),
            pl.BlockSpec((1, 1, C), lambda i: (0, 0, 0)),
        ],
        out_specs=[
            pl.BlockSpec((TN, C), lambda i: (i, 0)),
            pl.BlockSpec((TN,), lambda i: (i,)),
            pl.BlockSpec((1, 1, 1), lambda i: (i, 0, 0), memory_space=pltpu.SMEM),
        ],
        out_shape=[
            jax.ShapeDtypeStruct((N, C), jnp.float32),
            jax.ShapeDtypeStruct((N,), jnp.int32),
            jax.ShapeDtypeStruct((NI, 1, 1), jnp.float32),
        ],
    )(xt, emb, x2, e2)


def _sc_gather(emb, idx):
    info = plsc.get_sparse_core_info()
    nworkers = info.num_cores * info.num_subcores
    bpw = N // nworkers
    mesh = plsc.VectorSubcoreMesh(core_axis_name="c", subcore_axis_name="s")

    @functools.partial(
        pl.kernel, mesh=mesh,
        out_type=jax.ShapeDtypeStruct((N, D), jnp.float32),
        scratch_types=[
            pltpu.VMEM((bpw,), jnp.int32),
            pltpu.VMEM((bpw, D), jnp.float32),
            pltpu.SemaphoreType.DMA,
        ],
    )
    def k(emb_hbm, idx_hbm, out_hbm, idx_v, rows_v, sem):
        wid = jax.lax.axis_index("s") * info.num_cores + jax.lax.axis_index("c")
        base = wid * bpw
        pltpu.sync_copy(idx_hbm.at[pl.ds(base, bpw)], idx_v)
        pltpu.async_copy(emb_hbm.at[idx_v], rows_v, sem).wait()
        pltpu.sync_copy(rows_v, out_hbm.at[pl.ds(base, bpw)])

    return k(emb, idx)


def kernel(inputs, embedding):
    x = jnp.transpose(inputs, (0, 2, 1))
    xt = x[0]
    emb = embedding[0]
    x2 = jnp.sum(x * x, axis=-1, keepdims=True)[0]
    e2 = jnp.sum(embedding * embedding, axis=-1)[:, None, :]

    dist, idx_flat, lossp = _vq_distances(xt, emb, x2, e2)
    quant = _sc_gather(emb, idx_flat)

    out = jnp.transpose(quant)[None]
    encoding_indices = idx_flat[None]
    loss = (jnp.sum(lossp) * (COMMITMENT_COST / (N * D))).reshape(())
    distances = dist[None]
    return (out, encoding_indices, loss, distances)


# speedup vs baseline: 2.1081x; 1.0318x over previous
import functools

import jax
import jax.numpy as jnp
from jax.experimental import pallas as pl
from jax.experimental.pallas import tpu as pltpu
from jax.experimental.pallas import tpu_sc as plsc

COMMITMENT_COST = 0.25

N = 8192
C = 8192
D = 256

TN = 512
NI = N // TN


def _vq_body(x_ref, emb_ref, x2_ref, e2_ref, dist_ref, idx_ref, lossp_ref):
    x = x_ref[...]
    emb = emb_ref[...]

    xy = jax.lax.dot_general(
        x, emb, (((0,), (1,)), ((), ())),
        preferred_element_type=jnp.float32)
    x2 = x2_ref[...]
    e2 = e2_ref[0]
    d2 = jnp.maximum(x2 + e2 - 2.0 * xy, 0.0)
    vals = -jnp.sqrt(d2)
    dist_ref[...] = vals

    bmax = jnp.max(vals, axis=1, keepdims=True)
    col = jax.lax.broadcasted_iota(jnp.int32, (TN, C), 1)
    barg = jnp.min(jnp.where(vals == bmax, col, jnp.int32(C)),
                   axis=1, keepdims=True)
    idx_ref[...] = barg.reshape(TN)
    lossp_ref[0, 0, 0] = jnp.sum(bmax * bmax)


def _vq_distances(xdn, emb, x2, e2):
    return pl.pallas_call(
        _vq_body,
        grid=(NI,),
        in_specs=[
            pl.BlockSpec((D, TN), lambda i: (0, i)),
            pl.BlockSpec((C, D), lambda i: (0, 0)),
            pl.BlockSpec((TN, 1), lambda i: (i, 0)),
            pl.BlockSpec((1, 1, C), lambda i: (0, 0, 0)),
        ],
        out_specs=[
            pl.BlockSpec((TN, C), lambda i: (i, 0)),
            pl.BlockSpec((TN,), lambda i: (i,)),
            pl.BlockSpec((1, 1, 1), lambda i: (i, 0, 0), memory_space=pltpu.SMEM),
        ],
        out_shape=[
            jax.ShapeDtypeStruct((N, C), jnp.float32),
            jax.ShapeDtypeStruct((N,), jnp.int32),
            jax.ShapeDtypeStruct((NI, 1, 1), jnp.float32),
        ],
    )(xdn, emb, x2, e2)


def _sc_gather(emb, idx):
    info = plsc.get_sparse_core_info()
    nworkers = info.num_cores * info.num_subcores
    bpw = N // nworkers
    mesh = plsc.VectorSubcoreMesh(core_axis_name="c", subcore_axis_name="s")

    @functools.partial(
        pl.kernel, mesh=mesh,
        out_type=jax.ShapeDtypeStruct((N, D), jnp.float32),
        scratch_types=[
            pltpu.VMEM((bpw,), jnp.int32),
            pltpu.VMEM((bpw, D), jnp.float32),
            pltpu.SemaphoreType.DMA,
        ],
    )
    def k(emb_hbm, idx_hbm, out_hbm, idx_v, rows_v, sem):
        wid = jax.lax.axis_index("s") * info.num_cores + jax.lax.axis_index("c")
        base = wid * bpw
        pltpu.sync_copy(idx_hbm.at[pl.ds(base, bpw)], idx_v)
        pltpu.async_copy(emb_hbm.at[idx_v], rows_v, sem).wait()
        pltpu.sync_copy(rows_v, out_hbm.at[pl.ds(base, bpw)])

    return k(emb, idx)


def kernel(inputs, embedding):
    x = jnp.transpose(inputs, (0, 2, 1))
    emb = embedding[0]
    x2 = jnp.sum(x * x, axis=-1, keepdims=True)[0]
    e2 = jnp.sum(embedding * embedding, axis=-1)[:, None, :]

    dist, idx_flat, lossp = _vq_distances(inputs[0], emb, x2, e2)
    quant = _sc_gather(emb, idx_flat)

    out = jnp.transpose(quant)[None]
    encoding_indices = idx_flat[None]
    loss = (jnp.sum(lossp) * (COMMITMENT_COST / (N * D))).reshape(())
    distances = dist[None]
    return (out, encoding_indices, loss, distances)
